# baseline (speedup 1.0000x reference)
import jax
import jax.numpy as jnp
from jax import lax
from jax.experimental import pallas as pl
from jax.experimental.pallas import tpu as pltpu
from jax.experimental.pallas import tpu_sc as plsc

E = 1000000
D = 64
B = 16384
NC = 2
NS = 16
NW = NC * NS

NTILE_E = 32768
W = 512
NSLAB = NTILE_E // W
TAIL_E = 999936
FLUSH = 32
GROWS = 48
SH_ROWS = B + 8
DUMP_H = B + 4
DUMP_T = B + 4
SENTINEL = (32767 << 16) | (B + DUMP_T)

CHUNK = 2048
NCHUNK = B // CHUNK


def _bodyA(head_h, tail_h, ent_t_h, tailtab_h, sh_h, st_h,
           cb, matchbuf, tb, slab, gbh, gbt, jbh0, jbh1, jbt0, jbt1, st,
           sem_c, sem_s, sem_h, sem_t):
    iota = lax.iota(jnp.int32, 16)
    wid = lax.axis_index("s") * NC + lax.axis_index("c")

    @pl.when(wid <= 30)
    def _phase_a():
        for k in range(9):
            st[k] = 0
        jbhs = (jbh0, jbh1)
        jbts = (jbt0, jbt1)

        def reset_jb(ref, val):
            ref[pl.ds(0, 16)] = jnp.full((16,), val, jnp.int32)
            ref[pl.ds(16, 16)] = jnp.full((16,), val, jnp.int32)
            ref[pl.ds(32, 16)] = jnp.full((16,), val, jnp.int32)

        for p in range(2):
            reset_jb(jbhs[p], DUMP_H)
            reset_jb(jbts[p], DUMP_T)

        def fire_chunk(p, k):
            if k < NCHUNK:
                pltpu.async_copy(head_h.at[pl.ds(k * CHUNK, CHUNK)], cb.at[p], sem_c)
            elif k < 2 * NCHUNK:
                pltpu.async_copy(tail_h.at[pl.ds((k - NCHUNK) * CHUNK, CHUNK)],
                                 cb.at[p], sem_c)

        fire_chunk(0, 0)
        for k in range(2 * NCHUNK):
            p = k % 2
            pltpu.make_async_copy(head_h.at[pl.ds(0, CHUNK)], cb.at[p], sem_c).wait()
            fire_chunk(1 - p, k + 1)
            base_j = (k * CHUNK) if k < NCHUNK else (B + (k - NCHUNK) * CHUNK)

            def scanv(v, _):
                e = cb[p, pl.ds(v * 16, 16)]
                m = (e >> 15) == wid
                packed = ((e & 32767) << 16) | (base_j + v * 16 + iota)
                cnt = st[0]
                plsc.store_compressed(matchbuf.at[pl.ds(cnt, 16)], packed, mask=m)
                st[0] = cnt + plsc.all_reduce_population_count(m)[0]
                return 0

            lax.fori_loop(0, CHUNK // 16, scanv, 0)

        cnt = st[0]
        matchbuf[pl.ds(cnt, 16)] = jnp.full((16,), SENTINEL, jnp.int32)
        nv = (cnt + 15) >> 4

        nslab = jnp.where(wid == 30, 34, NSLAB)

        def fire_slab(p, b):
            @pl.when((b < nslab) & ((wid < 30) | (b < 33)))
            def _():
                off = wid * NTILE_E + b * W
                pltpu.async_copy(ent_t_h.at[:, pl.ds(off, W)], slab.at[p], sem_s)

            @pl.when((wid == 30) & (b == 33))
            def _():
                pltpu.async_copy(tailtab_h.at[:, :],
                                 slab.at[p, :, pl.ds(0, 128)], sem_s)

        def wait_slab(p, b):
            @pl.when((b < nslab) & ((wid < 30) | (b < 33)))
            def _():
                pltpu.make_async_copy(ent_t_h.at[:, pl.ds(0, W)],
                                      slab.at[p], sem_s).wait()

            @pl.when((wid == 30) & (b == 33))
            def _():
                pltpu.make_async_copy(tailtab_h.at[:, :],
                                      slab.at[p, :, pl.ds(0, 128)], sem_s).wait()

        def flush_h():
            ph = st[3]
            for p in range(2):
                @pl.when(ph == p)
                def _():
                    pltpu.async_copy(gbh.at[p], sh_h.at[jbhs[p]], sem_h)
                    st[5 + p] = 1
            for p in range(2):
                @pl.when((ph == 1 - p) & (st[5 + p] == 1))
                def _():
                    pltpu.make_async_copy(
                        gbh.at[p], sh_h.at[jbhs[p]], sem_h).wait()
                    st[5 + p] = 0
                    reset_jb(jbhs[p], DUMP_H)
            st[3] = 1 - ph
            st[1] = 0

        def flush_t():
            pt = st[4]
            for p in range(2):
                @pl.when(pt == p)
                def _():
                    pltpu.async_copy(gbt.at[p], st_h.at[jbts[p]], sem_t)
                    st[7 + p] = 1
            for p in range(2):
                @pl.when((pt == 1 - p) & (st[7 + p] == 1))
                def _():
                    pltpu.make_async_copy(
                        gbt.at[p], st_h.at[jbts[p]], sem_t).wait()
                    st[7 + p] = 0
                    reset_jb(jbts[p], DUMP_T)
            st[4] = 1 - pt
            st[2] = 0

        def process_slab(p, b):
            def scanm(v, _):
                pk = matchbuf[pl.ds(v * 16, 16)]
                m = (pk >> 25) == b
                plsc.store_compressed(tb.at[pl.ds(0, 16)], pk, mask=m)
                n = plsc.all_reduce_population_count(m)[0]

                def per_entry(k, _):
                    pp = tb[pl.ds(k, 16)][0]
                    er = (pp >> 16) & (W - 1)
                    j = pp & 65535
                    erv = er + jnp.zeros((16,), jnp.int32)
                    vals = [plsc.load_gather(slab.at[p], [iota + 16 * c, erv])
                            for c in range(4)]

                    @pl.when(j < B)
                    def _():
                        cur = st[1]
                        ph = st[3]
                        for c in range(4):
                            gbh[ph, cur, pl.ds(c * 16, 16)] = vals[c]
                        for p in range(2):
                            @pl.when(ph == p)
                            def _():
                                jv = jbhs[p][pl.ds(cur, 16)]
                                jbhs[p][pl.ds(cur, 16)] = jnp.where(
                                    iota == 0, j, jv)
                        st[1] = cur + 1
                        @pl.when(cur + 1 == FLUSH)
                        def _():
                            flush_h()

                    @pl.when(j >= B)
                    def _():
                        cur = st[2]
                        pt = st[4]
                        for c in range(4):
                            gbt[pt, cur, pl.ds(c * 16, 16)] = vals[c]
                        for p in range(2):
                            @pl.when(pt == p)
                            def _():
                                jv = jbts[p][pl.ds(cur, 16)]
                                jbts[p][pl.ds(cur, 16)] = jnp.where(
                                    iota == 0, j - B, jv)
                        st[2] = cur + 1
                        @pl.when(cur + 1 == FLUSH)
                        def _():
                            flush_t()

                    return 0

                lax.fori_loop(0, n, per_entry, 0)
                return 0

            lax.fori_loop(0, nv, scanm, 0)

        fire_slab(0, 0)

        def sweep(b2, _):
            b0 = b2 * 2
            wait_slab(0, b0)
            fire_slab(1, b0 + 1)
            process_slab(0, b0)
            wait_slab(1, b0 + 1)
            @pl.when(b0 + 2 < nslab)
            def _():
                fire_slab(0, b0 + 2)
            process_slab(1, b0 + 1)
            return 0

        lax.fori_loop(0, nslab >> 1, sweep, 0)

        @pl.when(st[1] > 0)
        def _():
            flush_h()

        @pl.when(st[2] > 0)
        def _():
            flush_t()

        for p in range(2):
            @pl.when(st[5 + p] == 1)
            def _():
                pltpu.make_async_copy(gbh.at[p], sh_h.at[jbhs[p]], sem_h).wait()

            @pl.when(st[7 + p] == 1)
            def _():
                pltpu.make_async_copy(gbt.at[p], st_h.at[jbts[p]], sem_t).wait()


def _bodyB(sh_h, st_h, relt_h, relidx_h, out_h,
           rows_h, rows_t, relslab, ridx, outv, semB):
    iota = lax.iota(jnp.int32, 16)
    wid = lax.axis_index("s") * NC + lax.axis_index("c")
    base = wid * 512
    pltpu.sync_copy(relt_h, relslab)
    pltpu.sync_copy(relidx_h.at[pl.ds(base, 512)], ridx.at[pl.ds(0, 512)])

    for ch in range(4):
        rb = base + ch * 128
        pltpu.sync_copy(sh_h.at[pl.ds(rb, 128), :], rows_h)
        pltpu.sync_copy(st_h.at[pl.ds(rb, 128), :], rows_t)

        def rowgroup(g, _):
            outvec = jnp.zeros((16,), jnp.float32)
            for i in range(16):
                row = g * 16 + i
                ri = ridx[pl.ds(ch * 128 + row, 16)][0]
                riv = ri + jnp.zeros((16,), jnp.int32)
                acc = None
                for c in range(4):
                    rv = plsc.load_gather(relslab, [iota + 16 * c, riv])
                    hv = rows_h[row, pl.ds(c * 16, 16)]
                    tv = rows_t[row, pl.ds(c * 16, 16)]
                    d = jnp.abs(hv + rv - tv)
                    acc = d if acc is None else acc + d
                s = jnp.sum(acc)
                outvec = jnp.where(iota == i, s, outvec)
            outv[pl.ds(ch * 128 + g * 16, 16)] = 0.0 - outvec
            return 0

        lax.fori_loop(0, 8, rowgroup, 0)

    pltpu.sync_copy(outv, out_h.at[pl.ds(base, 512)])


def _make_kernels():
    mesh = plsc.VectorSubcoreMesh(core_axis_name="c", subcore_axis_name="s")
    params = pltpu.CompilerParams(
        needs_layout_passes=False, use_tc_tiling_on_sc=True)
    ka = pl.kernel(
        _bodyA,
        out_type=(jax.ShapeDtypeStruct((SH_ROWS, 128), jnp.float32),
                  jax.ShapeDtypeStruct((SH_ROWS, 128), jnp.float32)),
        mesh=mesh,
        compiler_params=params,
        scratch_types=[
            pltpu.VMEM((2, CHUNK), jnp.int32),
            pltpu.VMEM((32800,), jnp.int32),
            pltpu.VMEM((32,), jnp.int32),
            pltpu.VMEM((2, D, W), jnp.float32),
            pltpu.VMEM((2, GROWS, 128), jnp.float32),
            pltpu.VMEM((2, GROWS, 128), jnp.float32),
            pltpu.VMEM((GROWS,), jnp.int32),
            pltpu.VMEM((GROWS,), jnp.int32),
            pltpu.VMEM((GROWS,), jnp.int32),
            pltpu.VMEM((GROWS,), jnp.int32),
            pltpu.SMEM((16,), jnp.int32),
            pltpu.SemaphoreType.DMA,
            pltpu.SemaphoreType.DMA,
            pltpu.SemaphoreType.DMA,
            pltpu.SemaphoreType.DMA,
        ],
    )
    kb = pl.kernel(
        _bodyB,
        out_type=jax.ShapeDtypeStruct((B,), jnp.float32),
        mesh=mesh,
        compiler_params=params,
        scratch_types=[
            pltpu.VMEM((128, 128), jnp.float32),
            pltpu.VMEM((128, 128), jnp.float32),
            pltpu.VMEM((D, 1024), jnp.float32),
            pltpu.VMEM((528,), jnp.int32),
            pltpu.VMEM((512,), jnp.float32),
            pltpu.SemaphoreType.DMA,
        ],
    )
    return ka, kb


_KA, _KB = _make_kernels()


@jax.jit
def _transe(head, rel, tail, ent_embedding, rel_embedding):
    ent_t = ent_embedding.T
    tailtab = jnp.pad(ent_embedding[TAIL_E:], ((0, 64), (0, 0))).T
    rel_t = jnp.pad(rel_embedding, ((0, 24), (0, 0))).T
    sh, st_ = _KA(head, tail, ent_t, tailtab)
    return _KB(sh, st_, rel_t, rel)


def kernel(head, rel, tail, ent_embedding, rel_embedding):
    return _transe(head, rel, tail, ent_embedding, rel_embedding).reshape(B, 1)
